# baseline (speedup 1.0000x reference)
import jax
import jax.numpy as jnp
from jax.experimental import pallas as pl

_HEADS = 4
_BLOCK_M = 1024


def _proj_kernel(x_ref, w_ref, b_ref, o_ref):
    o_ref[...] = (
        jnp.dot(x_ref[...], w_ref[...], preferred_element_type=jnp.float32)
        + b_ref[...]
    )


@jax.jit
def kernel(x, W, b):
    M, K = x.shape
    N = W.shape[1]
    b2 = b.reshape(1, N)
    out = pl.pallas_call(
        _proj_kernel,
        grid=(M // _BLOCK_M,),
        in_specs=[
            pl.BlockSpec((_BLOCK_M, K), lambda i: (i, 0)),
            pl.BlockSpec((K, N), lambda i: (0, 0)),
            pl.BlockSpec((1, N), lambda i: (0, 0)),
        ],
        out_specs=pl.BlockSpec((_BLOCK_M, N), lambda i: (i, 0)),
        out_shape=jax.ShapeDtypeStruct((M, N), jnp.float32),
    )(x, W, b2)
    return out.reshape(M, _HEADS, N // _HEADS)


# speedup vs baseline: 1.4194x; 1.4194x over previous
import jax
import jax.numpy as jnp
from jax.experimental import pallas as pl

_HEADS = 4
_BLOCK_M = 4096


def _proj_kernel(x_ref, w_ref, b_ref, o_ref):
    o_ref[...] = (
        jnp.dot(x_ref[...], w_ref[...], preferred_element_type=jnp.float32)
        + b_ref[...]
    )


@jax.jit
def kernel(x, W, b):
    M, K = x.shape
    N = W.shape[1]
    b2 = b.reshape(1, N)
    out = pl.pallas_call(
        _proj_kernel,
        grid=(M // _BLOCK_M,),
        in_specs=[
            pl.BlockSpec((_BLOCK_M, K), lambda i: (i, 0)),
            pl.BlockSpec((K, N), lambda i: (0, 0)),
            pl.BlockSpec((1, N), lambda i: (0, 0)),
        ],
        out_specs=pl.BlockSpec((_BLOCK_M, N), lambda i: (i, 0)),
        out_shape=jax.ShapeDtypeStruct((M, N), jnp.float32),
    )(x, W, b2)
    return out.reshape(M, _HEADS, N // _HEADS)
